# baseline (speedup 1.0000x reference)
import functools

import jax
import jax.numpy as jnp
from jax import lax
from jax.experimental import pallas as pl
from jax.experimental.pallas import tpu as pltpu
from jax.experimental.pallas import tpu_sc as plsc

C = 12
J = 23
D = 32
HM_H = 512
HM_W = 512
HMSZ = HM_H * HM_W
GRID = 104
HALF = 52
V = GRID * GRID * GRID
NC = 2
NS = 16
NW = NC * NS
PERW = V // NW
K = 104
NCH = PERW // K
L = 16
OBYTES = K * D * 4
IBYTES = C * K * 4


def _sc_gather_mean(tbl, idx):
    mesh = plsc.VectorSubcoreMesh(core_axis_name="c", subcore_axis_name="s")

    @functools.partial(
        pl.kernel,
        mesh=mesh,
        out_type=jax.ShapeDtypeStruct((V, D), jnp.float32),
        compiler_params=pltpu.CompilerParams(use_tc_tiling_on_sc=False),
        name="gather_mean",
        scratch_types=[
            pltpu.VMEM((C, K), jnp.int32),
            pltpu.VMEM((C, K), jnp.int32),
            pltpu.VMEM((C, K, D), jnp.float32),
            pltpu.VMEM((C, K, D), jnp.float32),
            pltpu.VMEM((K, D), jnp.float32),
            pltpu.VMEM((K, D), jnp.float32),
            pltpu.SemaphoreType.DMA,
            pltpu.SemaphoreType.DMA,
            pltpu.SemaphoreType.DMA,
            pltpu.SemaphoreType.DMA,
            pltpu.SemaphoreType.DMA,
            pltpu.SemaphoreType.DMA,
        ],
    )
    def body(tbl_hbm, idx_hbm, out_hbm, idx0, idx1, buf0, buf1, ob0, ob1,
             gs0, gs1, is0, is1, os0, os1):
        wid = lax.axis_index("s") * NC + lax.axis_index("c")
        idxs, bufs, obs = [idx0, idx1], [buf0, buf1], [ob0, ob1]
        gss, iss, oss = [gs0, gs1], [is0, is1], [os0, os1]

        def fire_gathers(b):
            for c in range(C):
                pltpu.async_copy(tbl_hbm.at[c].at[idxs[b].at[c]],
                                 bufs[b].at[c], gss[b])

        def drain_gathers(b):
            for c in range(C):
                pltpu.make_async_copy(tbl_hbm.at[0].at[pl.ds(0, K)],
                                      bufs[b].at[c], gss[b]).wait()

        pltpu.sync_copy(idx_hbm.at[wid, 0], idx0)
        fire_gathers(0)
        pltpu.async_copy(idx_hbm.at[wid, 1], idx1, is1)

        @pl.loop(0, NCH, step=2)
        def _pair(i):
            for b in range(2):
                ci = i + b
                nb = 1 - b

                drain_gathers(b)

                @pl.when(ci + 2 < NCH)
                def _pf():
                    pltpu.async_copy(idx_hbm.at[wid, ci + 2], idxs[b], iss[b])

                @pl.when(ci + 1 < NCH)
                def _fire():
                    pltpu.make_async_copy(idx_hbm.at[wid, 0], idxs[nb],
                                          iss[nb]).wait()
                    fire_gathers(nb)

                @pl.when(ci >= 2)
                def _wout():
                    pltpu.make_async_copy(out_hbm.at[pl.ds(0, K)], obs[b],
                                          oss[b]).wait()

                @pl.loop(0, K)
                def _red(k):
                    for d in range(D // L):
                        acc = bufs[b][0, k, pl.ds(L * d, L)]
                        for c in range(1, C):
                            acc = acc + bufs[b][c, k, pl.ds(L * d, L)]
                        obs[b][k, pl.ds(L * d, L)] = acc * (1.0 / C)

                pltpu.async_copy(
                    obs[b], out_hbm.at[pl.ds(wid * PERW + ci * K, K)], oss[b])

        for b in range(2):
            pltpu.make_async_copy(out_hbm.at[pl.ds(0, K)], obs[b],
                                  oss[b]).wait()

    return body(tbl, idx)


def kernel(heatmaps, center, reproLookup):
    ci = (center[0] / 2.0).astype(jnp.int32)
    sub = lax.dynamic_slice(
        reproLookup,
        (jnp.int32(0), ci[0] - HALF, ci[1] - HALF, ci[2] - HALF, jnp.int32(0)),
        (C, GRID, GRID, GRID, 2),
    )
    idx = sub[..., 1] * HM_W + sub[..., 0]
    idx = idx.reshape(C, NW, NCH, K).transpose(1, 2, 0, 3)

    tbl = jnp.pad(heatmaps[0], ((0, 0), (0, D - J), (0, 0), (0, 0)))
    tbl = jnp.transpose(tbl.reshape(C, D, HMSZ), (0, 2, 1))

    res = _sc_gather_mean(tbl, idx)
    out = res[:, :J].T.reshape(1, J, GRID, GRID, GRID)
    return out


# speedup vs baseline: 1.3289x; 1.3289x over previous
import functools

import jax
import jax.numpy as jnp
from jax import lax
from jax.experimental import pallas as pl
from jax.experimental.pallas import tpu as pltpu
from jax.experimental.pallas import tpu_sc as plsc

C = 12
J = 23
D = 32
HM_H = 512
HM_W = 512
HMSZ = HM_H * HM_W
GRID = 104
HALF = 52
V = GRID * GRID * GRID
NC = 2
NS = 16
NW = NC * NS
PERW = V // NW
K = 104
NCH = PERW // K
L = 16
OBYTES = K * D * 4
IBYTES = C * K * 4


def _sc_gather_mean(tbl, idx):
    mesh = plsc.VectorSubcoreMesh(core_axis_name="c", subcore_axis_name="s")

    @functools.partial(
        pl.kernel,
        mesh=mesh,
        out_type=jax.ShapeDtypeStruct((V, D), jnp.float32),
        compiler_params=pltpu.CompilerParams(use_tc_tiling_on_sc=False),
        name="gather_mean",
        scratch_types=[
            pltpu.VMEM((C, K), jnp.int32),
            pltpu.VMEM((C, K), jnp.int32),
            pltpu.VMEM((C, K, D), jnp.float32),
            pltpu.VMEM((C, K, D), jnp.float32),
            pltpu.VMEM((K, D), jnp.float32),
            pltpu.VMEM((K, D), jnp.float32),
            pltpu.SemaphoreType.DMA,
            pltpu.SemaphoreType.DMA,
            pltpu.SemaphoreType.DMA,
            pltpu.SemaphoreType.DMA,
            pltpu.SemaphoreType.DMA,
            pltpu.SemaphoreType.DMA,
        ],
    )
    def body(tbl_hbm, idx_hbm, out_hbm, idx0, idx1, buf0, buf1, ob0, ob1,
             gs0, gs1, is0, is1, os0, os1):
        wid = lax.axis_index("s") * NC + lax.axis_index("c")
        idxs, bufs, obs = [idx0, idx1], [buf0, buf1], [ob0, ob1]
        gss, iss, oss = [gs0, gs1], [is0, is1], [os0, os1]

        def fire_gathers(b):
            for c in range(C):
                pltpu.async_copy(tbl_hbm.at[idxs[b].at[c]], bufs[b].at[c],
                                 gss[b])

        def drain_gathers(b):
            for c in range(C):
                pltpu.make_async_copy(tbl_hbm.at[pl.ds(0, K)],
                                      bufs[b].at[c], gss[b]).wait()

        pltpu.sync_copy(idx_hbm.at[:, wid, 0], idx0)
        fire_gathers(0)
        pltpu.async_copy(idx_hbm.at[:, wid, 1], idx1, is1)

        @pl.loop(0, NCH, step=2)
        def _pair(i):
            for b in range(2):
                ci = i + b
                nb = 1 - b

                drain_gathers(b)

                @pl.when(ci + 2 < NCH)
                def _pf():
                    pltpu.async_copy(idx_hbm.at[:, wid, ci + 2], idxs[b],
                                     iss[b])

                @pl.when(ci + 1 < NCH)
                def _fire():
                    pltpu.make_async_copy(idx_hbm.at[:, wid, 0], idxs[nb],
                                          iss[nb]).wait()
                    fire_gathers(nb)

                @pl.when(ci >= 2)
                def _wout():
                    pltpu.make_async_copy(out_hbm.at[pl.ds(0, K)], obs[b],
                                          oss[b]).wait()

                @pl.loop(0, K)
                def _red(k):
                    for d in range(D // L):
                        acc = bufs[b][0, k, pl.ds(L * d, L)]
                        for c in range(1, C):
                            acc = acc + bufs[b][c, k, pl.ds(L * d, L)]
                        obs[b][k, pl.ds(L * d, L)] = acc * (1.0 / C)

                pltpu.async_copy(
                    obs[b], out_hbm.at[pl.ds(wid * PERW + ci * K, K)], oss[b])

        for b in range(2):
            pltpu.make_async_copy(out_hbm.at[pl.ds(0, K)], obs[b],
                                  oss[b]).wait()

    return body(tbl, idx)


def kernel(heatmaps, center, reproLookup):
    ci = (center[0] / 2.0).astype(jnp.int32)
    sub = lax.dynamic_slice(
        reproLookup,
        (jnp.int32(0), ci[0] - HALF, ci[1] - HALF, ci[2] - HALF, jnp.int32(0)),
        (C, GRID, GRID, GRID, 2),
    )
    coffs = (jnp.arange(C, dtype=jnp.int32) * HMSZ).reshape(C, 1, 1, 1)
    idx = sub[..., 1] * HM_W + sub[..., 0] + coffs
    idx = idx.reshape(C, NW, NCH, K)

    tbl = jnp.pad(heatmaps[0], ((0, 0), (0, D - J), (0, 0), (0, 0)))
    tbl = jnp.transpose(tbl, (0, 2, 3, 1)).reshape(C * HMSZ, D)

    res = _sc_gather_mean(tbl, idx)
    out = res[:, :J].T.reshape(1, J, GRID, GRID, GRID)
    return out
